# baseline (speedup 1.0000x reference)
import functools

import jax
import jax.numpy as jnp
from jax import lax
from jax.experimental import pallas as pl
from jax.experimental.pallas import tpu as pltpu
from jax.experimental.pallas import tpu_sc as plsc

D = 1024
LANES = 16
NC = 2
NS = 16
NW = NC * NS
CHUNK = 32
NBUF = 3


def _film_sc_body(x_hbm, ids_hbm, g_hbm, b_hbm, o_hbm,
                  ids_v, g_v, b_v, buf, in_sems, out_sems, gsem):
    wid = lax.axis_index("s") * NC + lax.axis_index("c")
    rows_per_w = x_hbm.shape[0] // NW
    n_chunks = rows_per_w // CHUNK
    rows_per_batch = x_hbm.shape[0] // ids_hbm.shape[0]
    base = wid * rows_per_w
    batch = (base // rows_per_batch).astype(jnp.int32)

    pltpu.sync_copy(ids_hbm, ids_v)
    pltpu.async_copy(g_hbm.at[ids_v], g_v, gsem).wait()
    pltpu.async_copy(b_hbm.at[ids_v], b_v, gsem).wait()

    def start_in(c, bi):
        return pltpu.async_copy(
            x_hbm.at[pl.ds(base + c * CHUNK, CHUNK), :],
            buf.at[bi], in_sems.at[bi])

    def start_out(c, bi):
        return pltpu.async_copy(
            buf.at[bi],
            o_hbm.at[pl.ds(base + c * CHUNK, CHUNK), :], out_sems.at[bi])

    def compute(bi):
        def dloop(k, _):
            col = pl.multiple_of(k * LANES, LANES)
            gv = g_v[batch, pl.ds(col, LANES)]
            bv = b_v[batch, pl.ds(col, LANES)]
            for r in range(CHUNK):
                buf[bi, r, pl.ds(col, LANES)] = (
                    gv * buf[bi, r, pl.ds(col, LANES)] + bv)
            return 0
        lax.fori_loop(0, D // LANES, dloop, 0)

    in_cp = [None] * NBUF
    out_cp = [None] * NBUF
    in_cp[0] = start_in(0, 0)
    if n_chunks > 1:
        in_cp[1] = start_in(1, 1)
    for c in range(n_chunks):
        bi = c % NBUF
        in_cp[bi].wait()
        compute(bi)
        out_cp[bi] = start_out(c, bi)
        nxt = c + 2
        if nxt < n_chunks:
            nbi = nxt % NBUF
            if out_cp[nbi] is not None:
                out_cp[nbi].wait()
            in_cp[nbi] = start_in(nxt, nbi)
    for cp in out_cp:
        if cp is not None:
            cp.wait()


@functools.partial(jax.jit, static_argnums=())
def _film_sc(x2d, ids, gamma, beta):
    mesh = plsc.VectorSubcoreMesh(core_axis_name="c", subcore_axis_name="s")
    return pl.kernel(
        _film_sc_body,
        out_type=jax.ShapeDtypeStruct(x2d.shape, x2d.dtype),
        mesh=mesh,
        scratch_types=[
            pltpu.VMEM((4,), jnp.int32),
            pltpu.VMEM((4, D), jnp.float32),
            pltpu.VMEM((4, D), jnp.float32),
            pltpu.VMEM((NBUF, CHUNK, D), jnp.float32),
            pltpu.SemaphoreType.DMA((NBUF,)),
            pltpu.SemaphoreType.DMA((NBUF,)),
            pltpu.SemaphoreType.DMA,
        ],
    )(x2d, ids, gamma, beta)


def kernel(x, condition_ids, gamma, beta):
    B, S, Dm = x.shape
    ids = condition_ids.astype(jnp.int32)
    out2d = _film_sc(x.reshape(B * S, Dm), ids, gamma, beta)
    return out2d.reshape(B, S, Dm)


# speedup vs baseline: 1.1358x; 1.1358x over previous
import functools

import jax
import jax.numpy as jnp
from jax import lax
from jax.experimental import pallas as pl
from jax.experimental.pallas import tpu as pltpu
from jax.experimental.pallas import tpu_sc as plsc

D = 1024
SEQ_BLOCK = 1024


def _gather_body(ids_hbm, g_hbm, b_hbm, go_hbm, bo_hbm, ids_v, gv, bv, sem):
    wid = lax.axis_index("s") * 2 + lax.axis_index("c")

    @pl.when(wid == 0)
    def _():
        pltpu.sync_copy(ids_hbm, ids_v)
        pltpu.async_copy(g_hbm.at[ids_v], gv, sem).wait()
        pltpu.async_copy(b_hbm.at[ids_v], bv, sem).wait()
        pltpu.sync_copy(gv, go_hbm)
        pltpu.sync_copy(bv, bo_hbm)


def _sc_gather(ids, gamma, beta):
    n, d = gamma.shape
    mesh = plsc.VectorSubcoreMesh(core_axis_name="c", subcore_axis_name="s")
    return pl.kernel(
        _gather_body,
        out_type=(
            jax.ShapeDtypeStruct((n, d), gamma.dtype),
            jax.ShapeDtypeStruct((n, d), beta.dtype),
        ),
        mesh=mesh,
        scratch_types=[
            pltpu.VMEM((n,), jnp.int32),
            pltpu.VMEM((n, d), jnp.float32),
            pltpu.VMEM((n, d), jnp.float32),
            pltpu.SemaphoreType.DMA,
        ],
    )(ids, gamma, beta)


def _film_body(x_ref, g_ref, b_ref, o_ref):
    o_ref[...] = g_ref[...] * x_ref[...] + b_ref[...]


@jax.jit
def _film(x, ids, gamma, beta):
    B, S, Dm = x.shape
    g_rows, b_rows = _sc_gather(ids, gamma, beta)
    g3 = g_rows.reshape(B, 1, Dm)
    b3 = b_rows.reshape(B, 1, Dm)
    return pl.pallas_call(
        _film_body,
        grid=(B, S // SEQ_BLOCK),
        in_specs=[
            pl.BlockSpec((1, SEQ_BLOCK, Dm), lambda b, s: (b, s, 0)),
            pl.BlockSpec((1, 1, Dm), lambda b, s: (b, 0, 0)),
            pl.BlockSpec((1, 1, Dm), lambda b, s: (b, 0, 0)),
        ],
        out_specs=pl.BlockSpec((1, SEQ_BLOCK, Dm), lambda b, s: (b, s, 0)),
        out_shape=jax.ShapeDtypeStruct((B, S, Dm), x.dtype),
        compiler_params=pltpu.CompilerParams(
            dimension_semantics=("parallel", "arbitrary"),
        ),
    )(x, g3, b3)


def kernel(x, condition_ids, gamma, beta):
    return _film(x, condition_ids.astype(jnp.int32), gamma, beta)
